# Initial kernel scaffold; baseline (speedup 1.0000x reference)
import jax
import jax.numpy as jnp
from jax.experimental import pallas as pl


def kernel(feat1, feat2, feat3, label1):
    raise NotImplementedError("write your pallas kernel here")


# speedup vs baseline: 8.7563x; 8.7563x over previous
import functools

import jax
import jax.numpy as jnp
from jax.experimental import pallas as pl
from jax.experimental.pallas import tpu as pltpu

N = 65536
D = 128
C = 512
MARGIN = 0.5
BLK = 2048
GRID = N // BLK


def _body(lbl_ref, f1_ref, f2_ref, f3_ref, out_ref, acc_ref, cnt_ref):
    i = pl.program_id(0)

    lbl = lbl_ref[0, 0, :]

    def norm(f):
        ss = jnp.sum(f * f, axis=1, keepdims=True)
        return f * jax.lax.rsqrt(ss)

    fstack = jnp.concatenate(
        [norm(f1_ref[...]), norm(f2_ref[...]), norm(f3_ref[...])], axis=1
    )

    classes = jax.lax.broadcasted_iota(jnp.int32, (BLK, C), 1)
    onehot = (lbl[:, None] == classes).astype(jnp.float32)

    part = jax.lax.dot_general(
        onehot, fstack, (((0,), (0,)), ((), ())),
        preferred_element_type=jnp.float32,
    )
    cnt = jnp.sum(onehot, axis=0)[None, :]

    @pl.when(i == 0)
    def _init():
        acc_ref[...] = part
        cnt_ref[...] = cnt

    @pl.when(i > 0)
    def _acc():
        acc_ref[...] += part
        cnt_ref[...] += cnt

    @pl.when(i == GRID - 1)
    def _finish():
        counts = cnt_ref[0, :]
        denom = jnp.maximum(counts, 1.0)[:, None]
        sums = acc_ref[...]
        c1 = sums[:, 0:D] / denom
        c2 = sums[:, D:2 * D] / denom
        c3 = sums[:, 2 * D:3 * D] / denom
        d = (jnp.sum((c1 - c2) ** 2, axis=1)
             + jnp.sum((c1 - c3) ** 2, axis=1)
             + jnp.sum((c2 - c3) ** 2, axis=1))
        per_class = jnp.where(counts > 0.0, jnp.maximum(d - MARGIN, 0.0), 0.0)
        out_ref[...] = jnp.sum(per_class)[None, None]


@functools.partial(jax.jit, static_argnames=())
def kernel(feat1, feat2, feat3, label1):
    lbl3 = label1.astype(jnp.int32).reshape(GRID, 1, BLK)
    out = pl.pallas_call(
        _body,
        grid=(GRID,),
        in_specs=[
            pl.BlockSpec((1, 1, BLK), lambda i: (i, 0, 0)),
            pl.BlockSpec((BLK, D), lambda i: (i, 0)),
            pl.BlockSpec((BLK, D), lambda i: (i, 0)),
            pl.BlockSpec((BLK, D), lambda i: (i, 0)),
        ],
        out_specs=pl.BlockSpec((1, 1), lambda i: (0, 0)),
        out_shape=jax.ShapeDtypeStruct((1, 1), jnp.float32),
        scratch_shapes=[
            pltpu.VMEM((C, 3 * D), jnp.float32),
            pltpu.VMEM((1, C), jnp.float32),
        ],
    )(lbl3, feat1, feat2, feat3)
    return out[0, 0]
